# Initial kernel scaffold; baseline (speedup 1.0000x reference)
import jax
import jax.numpy as jnp
from jax.experimental import pallas as pl


def kernel(x0, x1, x2):
    raise NotImplementedError("write your pallas kernel here")


# speedup vs baseline: 1.1395x; 1.1395x over previous
import functools

import jax
import jax.numpy as jnp
import numpy as np
from jax import lax
from jax.experimental import pallas as pl
from jax.experimental.pallas import tpu as pltpu
from jax.experimental.pallas import tpu_sc as plsc

_B, _N, _NC = 4, 25200, 80
_MAX_OBJ = 100
_NUM_BASE = 5
_ATTN_RES = 14
_MASK_RES = 56
_NDET = _B * _MAX_OBJ
_AP = _ATTN_RES * _ATTN_RES
_MP = _MASK_RES * _MASK_RES
_ROW = _NUM_BASE * _AP
_NPAD = 512
_DETS_PER_STEP = 40
_GRID = _NDET // _DETS_PER_STEP


def _consts_jax():
    rk = jax.random.key(1234)
    k1, k2, k3, k4, k5, k6 = jax.random.split(rk, 6)
    c = {}
    c["num_det"] = jax.random.randint(k1, (_B, 1), 0, _MAX_OBJ, dtype=jnp.int32)
    c["det_boxes"] = jax.random.normal(k2, (_B, _MAX_OBJ, 4), dtype=jnp.float32)
    c["det_scores"] = jax.random.normal(k3, (_B, _MAX_OBJ), dtype=jnp.float32)
    c["det_classes"] = jax.random.randint(k4, (_B, _MAX_OBJ), 0, _NC, dtype=jnp.int32)
    det_indices = jax.random.randint(k5, (_B, _MAX_OBJ), 0, _N, dtype=jnp.int32)
    c["pooled"] = jax.random.normal(
        k6, (_B, _MAX_OBJ, _NUM_BASE, _MASK_RES, _MASK_RES),
        dtype=jnp.float32).reshape(_NDET, _NUM_BASE, _MP)
    m1d = jax.image.resize(
        jnp.eye(_ATTN_RES, dtype=jnp.float32), (_MASK_RES, _ATTN_RES),
        method="bilinear")
    c["kmat"] = jnp.kron(m1d, m1d).T.astype(jnp.float32)
    bi = jnp.arange(_B, dtype=jnp.int32).repeat(_MAX_OBJ)
    fi = bi * _N + det_indices.reshape(-1)
    c["fi_pad"] = jnp.concatenate(
        [fi, jnp.zeros(_NPAD - _NDET, jnp.int32)])
    return c


def _build_consts():
    try:
        ctx = jax.default_device(jax.devices("cpu")[0])
    except Exception:
        import contextlib
        ctx = contextlib.nullcontext()
    with ctx:
        return {k: np.asarray(v) for k, v in _consts_jax().items()}


_C = _build_consts()

_SC_ROWS_PER_WORKER = 16


@functools.cache
def _make_sc_gather():
    mesh = plsc.VectorSubcoreMesh(core_axis_name="c", subcore_axis_name="s")

    @functools.partial(
        pl.kernel,
        mesh=mesh,
        out_type=jax.ShapeDtypeStruct((_NPAD, _ROW), jnp.float32),
        scratch_types=[
            pltpu.VMEM((_SC_ROWS_PER_WORKER,), jnp.int32),
            pltpu.VMEM((_SC_ROWS_PER_WORKER, _ROW), jnp.float32),
            pltpu.SemaphoreType.DMA,
        ],
    )
    def sc_gather(table_hbm, idx_hbm, out_hbm, idx_v, rows_v, sem):
        wid = lax.axis_index("s") * 2 + lax.axis_index("c")
        base = wid * _SC_ROWS_PER_WORKER
        pltpu.sync_copy(idx_hbm.at[pl.ds(base, _SC_ROWS_PER_WORKER)], idx_v)
        idx_vec = idx_v[...]
        for j in range(_SC_ROWS_PER_WORKER):
            pltpu.async_copy(
                table_hbm.at[pl.ds(idx_vec[j], 1)],
                rows_v.at[pl.ds(j, 1)],
                sem,
            )
        for j in range(_SC_ROWS_PER_WORKER):
            pltpu.make_async_copy(
                table_hbm.at[pl.ds(0, 1)], rows_v.at[pl.ds(j, 1)], sem
            ).wait()
        pltpu.sync_copy(rows_v, out_hbm.at[pl.ds(base, _SC_ROWS_PER_WORKER)])

    return sc_gather


def _tc_body(attn_ref, pooled_ref, kmat_ref, out_ref):
    a = attn_ref[...]
    k = kmat_ref[...]
    rs = [
        jnp.dot(a[:, j * _AP:(j + 1) * _AP], k,
                preferred_element_type=jnp.float32)
        for j in range(_NUM_BASE)
    ]
    m = rs[0]
    for j in range(1, _NUM_BASE):
        m = jnp.maximum(m, rs[j])
    es = [jnp.exp(rj - m) for rj in rs]
    s = es[0]
    for j in range(1, _NUM_BASE):
        s = s + es[j]
    acc = pooled_ref[:, 0, :] * es[0]
    for j in range(1, _NUM_BASE):
        acc = acc + pooled_ref[:, j, :] * es[j]
    out_ref[...] = jax.nn.sigmoid(acc / s)


def _tc_masks(attn_flat, pooled, kmat):
    return pl.pallas_call(
        _tc_body,
        grid=(_GRID,),
        in_specs=[
            pl.BlockSpec((_DETS_PER_STEP, _ROW), lambda i: (i, 0)),
            pl.BlockSpec((_DETS_PER_STEP, _NUM_BASE, _MP), lambda i: (i, 0, 0)),
            pl.BlockSpec((_AP, _MP), lambda i: (0, 0)),
        ],
        out_specs=pl.BlockSpec((_DETS_PER_STEP, _MP), lambda i: (i, 0)),
        out_shape=jax.ShapeDtypeStruct((_NDET, _MP), jnp.float32),
    )(attn_flat, pooled, kmat)


def kernel(x0, x1, x2):
    c = _C
    table = x1.reshape(_B * _N, _ROW)
    det_attn = _make_sc_gather()(table, jnp.asarray(c["fi_pad"]))
    masks = _tc_masks(det_attn, jnp.asarray(c["pooled"]),
                      jnp.asarray(c["kmat"]))
    return (
        jnp.asarray(c["num_det"]),
        jnp.asarray(c["det_boxes"]),
        jnp.asarray(c["det_scores"]),
        jnp.asarray(c["det_classes"]),
        masks.reshape(_B, _MAX_OBJ, _MP),
    )


# speedup vs baseline: 4.1203x; 3.6158x over previous
import functools

import jax
import jax.numpy as jnp
import numpy as np
from jax.experimental import pallas as pl
from jax.experimental.pallas import tpu as pltpu

_B, _N, _NC = 4, 25200, 80
_MAX_OBJ = 100
_NUM_BASE = 5
_ATTN_RES = 14
_MASK_RES = 56
_NDET = _B * _MAX_OBJ
_AP = _ATTN_RES * _ATTN_RES
_MP = _MASK_RES * _MASK_RES
_ROW = _NUM_BASE * _AP
_DETS_PER_STEP = 40
_GRID = _NDET // _DETS_PER_STEP
_SUBBLK = 8


def _consts_jax():
    rk = jax.random.key(1234)
    k1, k2, k3, k4, k5, k6 = jax.random.split(rk, 6)
    c = {}
    c["num_det"] = jax.random.randint(k1, (_B, 1), 0, _MAX_OBJ, dtype=jnp.int32)
    c["det_boxes"] = jax.random.normal(k2, (_B, _MAX_OBJ, 4), dtype=jnp.float32)
    c["det_scores"] = jax.random.normal(k3, (_B, _MAX_OBJ), dtype=jnp.float32)
    c["det_classes"] = jax.random.randint(k4, (_B, _MAX_OBJ), 0, _NC, dtype=jnp.int32)
    det_indices = jax.random.randint(k5, (_B, _MAX_OBJ), 0, _N, dtype=jnp.int32)
    c["pooled"] = jax.random.normal(
        k6, (_B, _MAX_OBJ, _NUM_BASE, _MASK_RES, _MASK_RES),
        dtype=jnp.float32).reshape(_NDET, _NUM_BASE, _MP)
    m1d = jax.image.resize(
        jnp.eye(_ATTN_RES, dtype=jnp.float32), (_MASK_RES, _ATTN_RES),
        method="bilinear")
    c["kmat"] = jnp.kron(m1d, m1d).T.astype(jnp.float32)
    c["det_bi"] = jnp.arange(_B, dtype=jnp.int32).repeat(_MAX_OBJ)
    c["det_di"] = det_indices.reshape(-1)
    return c


def _build_consts():
    try:
        ctx = jax.default_device(jax.devices("cpu")[0])
    except Exception:
        import contextlib
        ctx = contextlib.nullcontext()
    with ctx:
        return {k: np.asarray(v) for k, v in _consts_jax().items()}


_C = _build_consts()


def _body(bi_ref, d8_ref, sub_ref, *refs):
    x1_refs = refs[:_DETS_PER_STEP]
    pooled_ref, kmat_ref, out_ref, attn_ref = refs[_DETS_PER_STEP:]
    i = pl.program_id(0)
    iota8 = jax.lax.broadcasted_iota(jnp.int32, (_SUBBLK, 1), 0)
    for j in range(_DETS_PER_STEP):
        blk = x1_refs[j][0]
        sub = sub_ref[i * _DETS_PER_STEP + j]
        onehot = (iota8 == sub).astype(jnp.float32)
        attn_ref[j, :] = jnp.sum(blk * onehot, axis=0)
    a = attn_ref[...]
    k = kmat_ref[...]
    rs = [
        jnp.dot(a[:, j * _AP:(j + 1) * _AP], k,
                preferred_element_type=jnp.float32)
        for j in range(_NUM_BASE)
    ]
    m = rs[0]
    for j in range(1, _NUM_BASE):
        m = jnp.maximum(m, rs[j])
    es = [jnp.exp(rj - m) for rj in rs]
    s = es[0]
    for j in range(1, _NUM_BASE):
        s = s + es[j]
    acc = pooled_ref[:, 0, :] * es[0]
    for j in range(1, _NUM_BASE):
        acc = acc + pooled_ref[:, j, :] * es[j]
    out_ref[...] = jax.nn.sigmoid(acc / s)


def _x1_spec(j):
    return pl.BlockSpec(
        (1, _SUBBLK, _ROW),
        lambda i, bi, d8, sub: (bi[i * _DETS_PER_STEP + j],
                                d8[i * _DETS_PER_STEP + j], 0),
    )


@functools.cache
def _masks_call():
    grid_spec = pltpu.PrefetchScalarGridSpec(
        num_scalar_prefetch=3,
        grid=(_GRID,),
        in_specs=[_x1_spec(j) for j in range(_DETS_PER_STEP)] + [
            pl.BlockSpec((_DETS_PER_STEP, _NUM_BASE, _MP),
                         lambda i, *_: (i, 0, 0)),
            pl.BlockSpec((_AP, _MP), lambda i, *_: (0, 0)),
        ],
        out_specs=pl.BlockSpec((_DETS_PER_STEP, _MP), lambda i, *_: (i, 0)),
        scratch_shapes=[pltpu.VMEM((_DETS_PER_STEP, _ROW), jnp.float32)],
    )
    return pl.pallas_call(
        _body,
        grid_spec=grid_spec,
        out_shape=jax.ShapeDtypeStruct((_NDET, _MP), jnp.float32),
    )


def kernel(x0, x1, x2):
    c = _C
    bi = jnp.asarray(c["det_bi"])
    d8 = jnp.asarray(c["det_di"] // _SUBBLK)
    sub = jnp.asarray(c["det_di"] % _SUBBLK)
    masks = _masks_call()(
        bi, d8, sub,
        *([x1] * _DETS_PER_STEP),
        jnp.asarray(c["pooled"]),
        jnp.asarray(c["kmat"]),
    )
    return (
        jnp.asarray(c["num_det"]),
        jnp.asarray(c["det_boxes"]),
        jnp.asarray(c["det_scores"]),
        jnp.asarray(c["det_classes"]),
        masks.reshape(_B, _MAX_OBJ, _MP),
    )


# speedup vs baseline: 4.1803x; 1.0146x over previous
import functools

import jax
import jax.numpy as jnp
import numpy as np
from jax.experimental import pallas as pl
from jax.experimental.pallas import tpu as pltpu

_B, _N, _NC = 4, 25200, 80
_MAX_OBJ = 100
_NUM_BASE = 5
_ATTN_RES = 14
_MASK_RES = 56
_NDET = _B * _MAX_OBJ
_AP = _ATTN_RES * _ATTN_RES
_MP = _MASK_RES * _MASK_RES
_ROW = _NUM_BASE * _AP
_DETS_PER_STEP = 80
_GRID = _NDET // _DETS_PER_STEP
_SUBBLK = 8


def _consts_jax():
    rk = jax.random.key(1234)
    k1, k2, k3, k4, k5, k6 = jax.random.split(rk, 6)
    c = {}
    c["num_det"] = jax.random.randint(k1, (_B, 1), 0, _MAX_OBJ, dtype=jnp.int32)
    c["det_boxes"] = jax.random.normal(k2, (_B, _MAX_OBJ, 4), dtype=jnp.float32)
    c["det_scores"] = jax.random.normal(k3, (_B, _MAX_OBJ), dtype=jnp.float32)
    c["det_classes"] = jax.random.randint(k4, (_B, _MAX_OBJ), 0, _NC, dtype=jnp.int32)
    det_indices = jax.random.randint(k5, (_B, _MAX_OBJ), 0, _N, dtype=jnp.int32)
    c["pooled"] = jax.random.normal(
        k6, (_B, _MAX_OBJ, _NUM_BASE, _MASK_RES, _MASK_RES),
        dtype=jnp.float32).reshape(_NDET, _NUM_BASE, _MP)
    m1d = jax.image.resize(
        jnp.eye(_ATTN_RES, dtype=jnp.float32), (_MASK_RES, _ATTN_RES),
        method="bilinear")
    c["kmat"] = jnp.kron(m1d, m1d).T.astype(jnp.float32)
    c["det_bi"] = jnp.arange(_B, dtype=jnp.int32).repeat(_MAX_OBJ)
    c["det_di"] = det_indices.reshape(-1)
    return c


def _build_consts():
    try:
        ctx = jax.default_device(jax.devices("cpu")[0])
    except Exception:
        import contextlib
        ctx = contextlib.nullcontext()
    with ctx:
        return {k: np.asarray(v) for k, v in _consts_jax().items()}


_C = _build_consts()


def _body(bi_ref, d8_ref, sub_ref, *refs):
    x1_refs = refs[:_DETS_PER_STEP]
    pooled_ref, kmat_ref, out_ref, attn_ref = refs[_DETS_PER_STEP:]
    i = pl.program_id(0)
    iota8 = jax.lax.broadcasted_iota(jnp.int32, (_SUBBLK, 1), 0)
    for j in range(_DETS_PER_STEP):
        blk = x1_refs[j][0]
        sub = sub_ref[i * _DETS_PER_STEP + j]
        onehot = (iota8 == sub).astype(jnp.float32)
        attn_ref[j, :] = jnp.sum(blk * onehot, axis=0)
    a = attn_ref[...]
    k = kmat_ref[...]
    rs = [
        jnp.dot(a[:, j * _AP:(j + 1) * _AP], k,
                preferred_element_type=jnp.float32)
        for j in range(_NUM_BASE)
    ]
    m = rs[0]
    for j in range(1, _NUM_BASE):
        m = jnp.maximum(m, rs[j])
    es = [jnp.exp(rj - m) for rj in rs]
    s = es[0]
    for j in range(1, _NUM_BASE):
        s = s + es[j]
    acc = pooled_ref[:, 0, :] * es[0]
    for j in range(1, _NUM_BASE):
        acc = acc + pooled_ref[:, j, :] * es[j]
    out_ref[...] = jax.nn.sigmoid(acc / s)


def _x1_spec(j):
    return pl.BlockSpec(
        (1, _SUBBLK, _ROW),
        lambda i, bi, d8, sub: (bi[i * _DETS_PER_STEP + j],
                                d8[i * _DETS_PER_STEP + j], 0),
    )


@functools.cache
def _masks_call():
    grid_spec = pltpu.PrefetchScalarGridSpec(
        num_scalar_prefetch=3,
        grid=(_GRID,),
        in_specs=[_x1_spec(j) for j in range(_DETS_PER_STEP)] + [
            pl.BlockSpec((_DETS_PER_STEP, _NUM_BASE, _MP),
                         lambda i, *_: (i, 0, 0)),
            pl.BlockSpec((_AP, _MP), lambda i, *_: (0, 0)),
        ],
        out_specs=pl.BlockSpec((_DETS_PER_STEP, _MP), lambda i, *_: (i, 0)),
        scratch_shapes=[pltpu.VMEM((_DETS_PER_STEP, _ROW), jnp.float32)],
    )
    return pl.pallas_call(
        _body,
        grid_spec=grid_spec,
        out_shape=jax.ShapeDtypeStruct((_NDET, _MP), jnp.float32),
    )


def kernel(x0, x1, x2):
    c = _C
    bi = jnp.asarray(c["det_bi"])
    d8 = jnp.asarray(c["det_di"] // _SUBBLK)
    sub = jnp.asarray(c["det_di"] % _SUBBLK)
    masks = _masks_call()(
        bi, d8, sub,
        *([x1] * _DETS_PER_STEP),
        jnp.asarray(c["pooled"]),
        jnp.asarray(c["kmat"]),
    )
    return (
        jnp.asarray(c["num_det"]),
        jnp.asarray(c["det_boxes"]),
        jnp.asarray(c["det_scores"]),
        jnp.asarray(c["det_classes"]),
        masks.reshape(_B, _MAX_OBJ, _MP),
    )


# speedup vs baseline: 4.2495x; 1.0166x over previous
import functools

import jax
import jax.numpy as jnp
import numpy as np
from jax.experimental import pallas as pl
from jax.experimental.pallas import tpu as pltpu

_B, _N, _NC = 4, 25200, 80
_MAX_OBJ = 100
_NUM_BASE = 5
_ATTN_RES = 14
_MASK_RES = 56
_NDET = _B * _MAX_OBJ
_AP = _ATTN_RES * _ATTN_RES
_MP = _MASK_RES * _MASK_RES
_ROW = _NUM_BASE * _AP
_DETS_PER_STEP = 80
_GRID = _NDET // _DETS_PER_STEP
_SUBBLK = 8


def _consts_jax():
    rk = jax.random.key(1234)
    k1, k2, k3, k4, k5, k6 = jax.random.split(rk, 6)
    c = {}
    c["num_det"] = jax.random.randint(k1, (_B, 1), 0, _MAX_OBJ, dtype=jnp.int32)
    c["det_boxes"] = jax.random.normal(k2, (_B, _MAX_OBJ, 4), dtype=jnp.float32)
    c["det_scores"] = jax.random.normal(k3, (_B, _MAX_OBJ), dtype=jnp.float32)
    c["det_classes"] = jax.random.randint(k4, (_B, _MAX_OBJ), 0, _NC, dtype=jnp.int32)
    det_indices = jax.random.randint(k5, (_B, _MAX_OBJ), 0, _N, dtype=jnp.int32)
    c["pooled"] = jax.random.normal(
        k6, (_B, _MAX_OBJ, _NUM_BASE, _MASK_RES, _MASK_RES),
        dtype=jnp.float32).reshape(_NDET, _NUM_BASE, _MP)
    m1d = jax.image.resize(
        jnp.eye(_ATTN_RES, dtype=jnp.float32), (_MASK_RES, _ATTN_RES),
        method="bilinear")
    c["kmat"] = jnp.kron(m1d, m1d).T.astype(jnp.float32)
    c["det_bi"] = jnp.arange(_B, dtype=jnp.int32).repeat(_MAX_OBJ)
    c["det_di"] = det_indices.reshape(-1)
    return c


def _build_consts():
    try:
        ctx = jax.default_device(jax.devices("cpu")[0])
    except Exception:
        import contextlib
        ctx = contextlib.nullcontext()
    with ctx:
        return {k: np.asarray(v) for k, v in _consts_jax().items()}


_C = _build_consts()


def _body(bi_ref, di_ref, x1_ref, pooled_ref, kmat_ref, out_ref,
          attn_all, sem):
    i = pl.program_id(0)
    @pl.when(i == 0)
    def _():
        for g in range(_NDET):
            pltpu.make_async_copy(
                x1_ref.at[bi_ref[g], pl.ds(di_ref[g], 1)],
                attn_all.at[pl.ds(g, 1)],
                sem.at[g // _DETS_PER_STEP],
            ).start()
    for g in range(_DETS_PER_STEP):
        pltpu.make_async_copy(
            x1_ref.at[0, pl.ds(0, 1)],
            attn_all.at[pl.ds(g, 1)],
            sem.at[i],
        ).wait()
    a = attn_all[pl.ds(i * _DETS_PER_STEP, _DETS_PER_STEP), :]
    k = kmat_ref[...]
    rs = [
        jnp.dot(a[:, j * _AP:(j + 1) * _AP], k,
                preferred_element_type=jnp.float32)
        for j in range(_NUM_BASE)
    ]
    m = rs[0]
    for j in range(1, _NUM_BASE):
        m = jnp.maximum(m, rs[j])
    es = [jnp.exp(rj - m) for rj in rs]
    s = es[0]
    for j in range(1, _NUM_BASE):
        s = s + es[j]
    acc = pooled_ref[:, 0, :] * es[0]
    for j in range(1, _NUM_BASE):
        acc = acc + pooled_ref[:, j, :] * es[j]
    out_ref[...] = jax.nn.sigmoid(acc / s)


@functools.cache
def _masks_call():
    grid_spec = pltpu.PrefetchScalarGridSpec(
        num_scalar_prefetch=2,
        grid=(_GRID,),
        in_specs=[
            pl.BlockSpec(memory_space=pltpu.MemorySpace.HBM),
            pl.BlockSpec((_DETS_PER_STEP, _NUM_BASE, _MP),
                         lambda i, *_: (i, 0, 0)),
            pl.BlockSpec((_AP, _MP), lambda i, *_: (0, 0)),
        ],
        out_specs=pl.BlockSpec((_DETS_PER_STEP, _MP), lambda i, *_: (i, 0)),
        scratch_shapes=[
            pltpu.VMEM((_NDET, _ROW), jnp.float32),
            pltpu.SemaphoreType.DMA((_GRID,)),
        ],
    )
    return pl.pallas_call(
        _body,
        grid_spec=grid_spec,
        out_shape=jax.ShapeDtypeStruct((_NDET, _MP), jnp.float32),
    )


def kernel(x0, x1, x2):
    c = _C
    bi = jnp.asarray(c["det_bi"])
    di = jnp.asarray(c["det_di"])
    masks = _masks_call()(
        bi, di, x1,
        jnp.asarray(c["pooled"]),
        jnp.asarray(c["kmat"]),
    )
    return (
        jnp.asarray(c["num_det"]),
        jnp.asarray(c["det_boxes"]),
        jnp.asarray(c["det_scores"]),
        jnp.asarray(c["det_classes"]),
        masks.reshape(_B, _MAX_OBJ, _MP),
    )
